# baseline (speedup 1.0000x reference)
import functools

import jax
import jax.numpy as jnp
from jax import lax
from jax.experimental import pallas as pl
from jax.experimental.pallas import tpu as pltpu
from jax.experimental.pallas import tpu_sc as plsc

N_NODES = 10000
D = 128
DP = D // 2
E = 320000

NC = 2
NS = 16
NW = NC * NS
L = 16

EPW = E // NW
K_CH = 125
B_CH = EPW // K_CH
ZR = 80
NZC = N_NODES // ZR

def _deg_body(dst_hbm, ew_hbm, out_hbm, dst_v, ew_v, deg_v):
    cid = lax.axis_index("c")
    sid = lax.axis_index("s")
    wid = sid * NC + cid

    def zero_body(i, _):
        deg_v[pl.ds(i * L, L)] = jnp.zeros((L,), jnp.float32)
        return 0

    lax.fori_loop(0, N_NODES // L, zero_body, 0)

    pltpu.sync_copy(dst_hbm.at[pl.ds(wid * EPW, EPW)], dst_v)
    pltpu.sync_copy(ew_hbm.at[pl.ds(wid * EPW, EPW)], ew_v)

    def acc_body(i, _):
        idx = dst_v[pl.ds(i * L, L)]
        w = ew_v[pl.ds(i * L, L)]
        plsc.addupdate_scatter(deg_v, [idx], w)
        return 0

    lax.fori_loop(0, EPW // L, acc_body, 0)

    pltpu.sync_copy(deg_v, out_hbm.at[pl.ds(wid * N_NODES, N_NODES)])


@functools.cache
def _deg_kernel():
    mesh = plsc.VectorSubcoreMesh(core_axis_name="c", subcore_axis_name="s",
                                  num_cores=NC, num_subcores=NS)
    return pl.kernel(
        _deg_body,
        out_type=jax.ShapeDtypeStruct((NW * N_NODES,), jnp.float32),
        mesh=mesh,
        compiler_params=pltpu.CompilerParams(needs_layout_passes=False),
        scratch_types=[
            pltpu.VMEM((EPW,), jnp.int32),
            pltpu.VMEM((EPW,), jnp.float32),
            pltpu.VMEM((N_NODES,), jnp.float32),
        ],
    )


def _agg_body(src_hbm, dst_hbm, ew_hbm, hp_hbm, out_hbm,
              src_v, didx0_v, didx1_v, didx2_v, ewc0_v, ewc1_v, ewc2_v,
              grow0_v, grow1_v, grow2_v, sidx0_v, sidx1_v, srow0_v, srow1_v,
              acc_sh, gsem0, gsem1, gsem2, ssem0, ssem1):
    cid = lax.axis_index("c")
    sid = lax.axis_index("s")
    wid = sid * NC + cid

    def zb_body(i, _):
        for k in range(D // L):
            srow0_v[i, pl.ds(k * L, L)] = jnp.zeros((L,), jnp.float32)
        return 0

    lax.fori_loop(0, ZR, zb_body, 0)

    for t in range((NZC + NS - 1) // NS):
        c = sid + NS * t

        @pl.when(c < NZC)
        def _():
            pltpu.sync_copy(srow0_v, acc_sh.at[pl.ds(c * ZR, ZR)])

    pltpu.sync_copy(src_hbm.at[pl.ds(wid * EPW, EPW)], src_v)

    plsc.subcore_barrier()

    grow = (grow0_v, grow1_v, grow2_v)
    didx = (didx0_v, didx1_v, didx2_v)
    ewc = (ewc0_v, ewc1_v, ewc2_v)
    gsem = (gsem0, gsem1, gsem2)
    sidx = (sidx0_v, sidx1_v)
    srow = (srow0_v, srow1_v)
    ssem = (ssem0, ssem1)

    def gather_start(j, p):
        pltpu.async_copy(hp_hbm.at[src_v.at[pl.ds(j * B_CH, B_CH)]],
                         grow[p], gsem[p])
        pltpu.async_copy(dst_hbm.at[pl.ds(wid * EPW + j * B_CH, B_CH)],
                         didx[p], gsem[p])
        pltpu.async_copy(ew_hbm.at[pl.ds(wid * EPW + j * B_CH, B_CH)],
                         ewc[p], gsem[p])

    def gather_wait(j, p):
        pltpu.make_async_copy(hp_hbm.at[src_v.at[pl.ds(j * B_CH, B_CH)]],
                              grow[p], gsem[p]).wait()
        pltpu.make_async_copy(dst_hbm.at[pl.ds(wid * EPW + j * B_CH, B_CH)],
                              didx[p], gsem[p]).wait()
        pltpu.make_async_copy(ew_hbm.at[pl.ds(wid * EPW + j * B_CH, B_CH)],
                              ewc[p], gsem[p]).wait()

    def scale(p, m):
        for t in range(B_CH // L):
            sidx[m][pl.ds(t * L, L)] = didx[p][pl.ds(t * L, L)]

        def group_body(g, _):
            ew16 = ewc[p][pl.ds(g * L, L)]
            base = g * L
            for i in range(L):
                s = ew16[i]
                for k in range(DP // L):
                    w = grow[p][base + i, pl.ds(k * L, L)]
                    bf = plsc.bitcast(w, jnp.bfloat16)
                    a, b = plsc.unpack(bf, format=plsc.PackFormat.INTERLEAVED)
                    srow[m][base + i, pl.ds(k * L, L)] = a * s
                    srow[m][base + i, pl.ds(DP + k * L, L)] = b * s
            return 0

        lax.fori_loop(0, B_CH // L, group_body, 0)

    def scatter_start(m):
        pltpu.async_copy(srow[m], acc_sh.at[sidx[m]], ssem[m], add=True)

    def scatter_wait(m):
        pltpu.make_async_copy(srow[m], acc_sh.at[sidx[m]], ssem[m]).wait()

    def step(j, p, m, wait_prev, next_j):
        gather_wait(j, p)
        if wait_prev:
            scatter_wait(m)
        scale(p, m)
        scatter_start(m)
        if next_j is not None:
            gather_start(next_j, (p + 2) % 3)

    gather_start(0, 0)
    gather_start(1, 1)
    step(0, 0, 0, False, 2)
    step(1, 1, 1, False, 3)

    def ring_body(g, _):
        j = 6 * g + 2
        step(j, 2, 0, True, j + 2)
        step(j + 1, 0, 1, True, j + 3)
        step(j + 2, 1, 0, True, j + 4)
        step(j + 3, 2, 1, True, j + 5)
        step(j + 4, 0, 0, True, j + 6)
        step(j + 5, 1, 1, True, j + 7)
        return 0

    lax.fori_loop(0, (K_CH - 5) // 6, ring_body, 0)
    step(K_CH - 3, 2, 0, True, K_CH - 1)
    step(K_CH - 2, 0, 1, True, None)
    step(K_CH - 1, 1, 0, True, None)
    scatter_wait(1)
    scatter_wait(0)

    plsc.subcore_barrier()

    for t in range((NZC + NS - 1) // NS):
        c = sid + NS * t

        @pl.when(c < NZC)
        def _():
            pltpu.sync_copy(acc_sh.at[pl.ds(c * ZR, ZR)],
                            out_hbm.at[pl.ds(cid * N_NODES + c * ZR, ZR)])


@functools.cache
def _agg_kernel():
    mesh = plsc.VectorSubcoreMesh(core_axis_name="c", subcore_axis_name="s",
                                  num_cores=NC, num_subcores=NS)
    return pl.kernel(
        _agg_body,
        out_type=jax.ShapeDtypeStruct((NC * N_NODES, D), jnp.float32),
        mesh=mesh,
        compiler_params=pltpu.CompilerParams(needs_layout_passes=False,
                                             use_tc_tiling_on_sc=False),
        scratch_types=[
            pltpu.VMEM((EPW,), jnp.int32),
            pltpu.VMEM((B_CH,), jnp.int32),
            pltpu.VMEM((B_CH,), jnp.int32),
            pltpu.VMEM((B_CH,), jnp.int32),
            pltpu.VMEM((B_CH,), jnp.float32),
            pltpu.VMEM((B_CH,), jnp.float32),
            pltpu.VMEM((B_CH,), jnp.float32),
            pltpu.VMEM((B_CH, DP), jnp.int32),
            pltpu.VMEM((B_CH, DP), jnp.int32),
            pltpu.VMEM((B_CH, DP), jnp.int32),
            pltpu.VMEM((B_CH,), jnp.int32),
            pltpu.VMEM((B_CH,), jnp.int32),
            pltpu.VMEM((B_CH, D), jnp.float32),
            pltpu.VMEM((B_CH, D), jnp.float32),
            pltpu.VMEM_SHARED((N_NODES, D), jnp.float32),
            pltpu.SemaphoreType.DMA,
            pltpu.SemaphoreType.DMA,
            pltpu.SemaphoreType.DMA,
            pltpu.SemaphoreType.DMA,
            pltpu.SemaphoreType.DMA,
        ],
    )


BLK = 2000


def _tcA_body(degp_ref, x_ref, w_ref, hp_ref, dinv_ref):
    deg = jnp.sum(degp_ref[...], axis=1) + 2.0
    dinv = jnp.where(deg > 0, lax.rsqrt(jnp.maximum(deg, 1e-30)), 0.0)
    h = jnp.dot(x_ref[...], w_ref[...], preferred_element_type=jnp.float32)
    u = lax.bitcast_convert_type(h * dinv[:, None], jnp.uint32)
    r = (u + 0x8000) >> 16
    packed = r[:, :DP] | (r[:, DP:] << 16)
    hp_ref[...] = lax.bitcast_convert_type(packed, jnp.int32)
    dinv_ref[...] = dinv[:, None]


_tcA = pl.pallas_call(
    _tcA_body,
    grid=(N_NODES // BLK,),
    in_specs=[
        pl.BlockSpec((BLK, NW), lambda i: (i, 0)),
        pl.BlockSpec((BLK, D), lambda i: (i, 0)),
        pl.BlockSpec((D, D), lambda i: (0, 0)),
    ],
    out_specs=[
        pl.BlockSpec((BLK, DP), lambda i: (i, 0)),
        pl.BlockSpec((BLK, 1), lambda i: (i, 0)),
    ],
    out_shape=[
        jax.ShapeDtypeStruct((N_NODES, DP), jnp.int32),
        jax.ShapeDtypeStruct((N_NODES, 1), jnp.float32),
    ],
)


def _tcC_body(p_ref, hp_ref, dinv_ref, b1_ref, wfc_ref, bfc_ref, out_ref):
    acc = p_ref[0] + p_ref[1]
    pu = lax.bitcast_convert_type(hp_ref[...], jnp.uint32)
    lo = lax.bitcast_convert_type(pu << 16, jnp.float32)
    hi = lax.bitcast_convert_type(pu & jnp.uint32(0xFFFF0000), jnp.float32)
    hrows = jnp.concatenate([lo, hi], axis=1)
    dinv = dinv_ref[...]
    pre = dinv * acc + (2.0 * dinv) * hrows + b1_ref[...]
    r = jnp.maximum(pre, 0.0)
    out_ref[...] = jnp.dot(r, wfc_ref[...],
                           preferred_element_type=jnp.float32) + bfc_ref[...]


_tcC = pl.pallas_call(
    _tcC_body,
    grid=(N_NODES // BLK,),
    in_specs=[
        pl.BlockSpec((NC, BLK, D), lambda i: (0, i, 0)),
        pl.BlockSpec((BLK, DP), lambda i: (i, 0)),
        pl.BlockSpec((BLK, 1), lambda i: (i, 0)),
        pl.BlockSpec((1, D), lambda i: (0, 0)),
        pl.BlockSpec((D, 1), lambda i: (0, 0)),
        pl.BlockSpec((1, 1), lambda i: (0, 0)),
    ],
    out_specs=pl.BlockSpec((BLK, 1), lambda i: (i, 0)),
    out_shape=jax.ShapeDtypeStruct((N_NODES, 1), jnp.float32),
)


def kernel(x, edge_index, edge_attr, W1, b1, Wfc, bfc):
    ei = edge_index.astype(jnp.int32)
    src = ei[0]
    dst = ei[1]
    ew = edge_attr.astype(jnp.float32)

    degp = _deg_kernel()(dst, ew)
    degp_t = degp.reshape(NW, N_NODES).T

    hp, dinv = _tcA(degp_t, x.astype(jnp.float32), W1)

    parts = _agg_kernel()(src, dst, ew, hp)
    parts = parts.reshape(NC, N_NODES, D)

    out = _tcC(parts, hp, dinv, b1.reshape(1, D), Wfc, bfc.reshape(1, 1))
    return out


# speedup vs baseline: 1.7322x; 1.7322x over previous
import functools

import jax
import jax.numpy as jnp
from jax import lax
from jax.experimental import pallas as pl
from jax.experimental.pallas import tpu as pltpu
from jax.experimental.pallas import tpu_sc as plsc

N_NODES = 10000
D = 128
E = 320000

NC = 2
NS = 16
NW = NC * NS
L = 16

EPW = E // NW
K_CH = 125
B_CH = EPW // K_CH
ZR = 80
NZC = N_NODES // ZR


def _deg_body(dst_hbm, ew_hbm, out_hbm, dst_v, ew_v, deg_v):
    cid = lax.axis_index("c")
    sid = lax.axis_index("s")
    wid = sid * NC + cid

    def zero_body(i, _):
        deg_v[pl.ds(i * L, L)] = jnp.zeros((L,), jnp.float32)
        return 0

    lax.fori_loop(0, N_NODES // L, zero_body, 0)

    pltpu.sync_copy(dst_hbm.at[pl.ds(wid * EPW, EPW)], dst_v)
    pltpu.sync_copy(ew_hbm.at[pl.ds(wid * EPW, EPW)], ew_v)

    def acc_body(i, _):
        idx = dst_v[pl.ds(i * L, L)]
        w = ew_v[pl.ds(i * L, L)]
        plsc.addupdate_scatter(deg_v, [idx], w)
        return 0

    lax.fori_loop(0, EPW // L, acc_body, 0)

    pltpu.sync_copy(deg_v, out_hbm.at[pl.ds(wid * N_NODES, N_NODES)])


@functools.cache
def _deg_kernel():
    mesh = plsc.VectorSubcoreMesh(core_axis_name="c", subcore_axis_name="s",
                                  num_cores=NC, num_subcores=NS)
    return pl.kernel(
        _deg_body,
        out_type=jax.ShapeDtypeStruct((NW * N_NODES,), jnp.float32),
        mesh=mesh,
        compiler_params=pltpu.CompilerParams(needs_layout_passes=False),
        scratch_types=[
            pltpu.VMEM((EPW,), jnp.int32),
            pltpu.VMEM((EPW,), jnp.float32),
            pltpu.VMEM((N_NODES,), jnp.float32),
        ],
    )


def _agg_body(src_hbm, dst_hbm, ew_hbm, hp_hbm, out_hbm,
              src_v, didx0_v, didx1_v, didx2_v, ewc0_v, ewc1_v, ewc2_v,
              rows0_v, rows1_v, rows2_v, acc_sh,
              gsem0, gsem1, gsem2, ssem0, ssem1, ssem2):
    cid = lax.axis_index("c")
    sid = lax.axis_index("s")
    wid = sid * NC + cid

    def zb_body(i, _):
        for k in range(D // L):
            rows0_v[i, pl.ds(k * L, L)] = jnp.zeros((L,), jnp.float32)
        return 0

    lax.fori_loop(0, ZR, zb_body, 0)

    for t in range((NZC + NS - 1) // NS):
        c = sid + NS * t

        @pl.when(c < NZC)
        def _():
            pltpu.sync_copy(rows0_v, acc_sh.at[pl.ds(c * ZR, ZR)])

    pltpu.sync_copy(src_hbm.at[pl.ds(wid * EPW, EPW)], src_v)

    plsc.subcore_barrier()

    rows = (rows0_v, rows1_v, rows2_v)
    didx = (didx0_v, didx1_v, didx2_v)
    ewc = (ewc0_v, ewc1_v, ewc2_v)
    gsem = (gsem0, gsem1, gsem2)
    ssem = (ssem0, ssem1, ssem2)

    def gather_start(j, p):
        pltpu.async_copy(hp_hbm.at[src_v.at[pl.ds(j * B_CH, B_CH)]],
                         rows[p], gsem[p])
        pltpu.async_copy(dst_hbm.at[pl.ds(wid * EPW + j * B_CH, B_CH)],
                         didx[p], gsem[p])
        pltpu.async_copy(ew_hbm.at[pl.ds(wid * EPW + j * B_CH, B_CH)],
                         ewc[p], gsem[p])

    def gather_wait(j, p):
        pltpu.make_async_copy(hp_hbm.at[src_v.at[pl.ds(j * B_CH, B_CH)]],
                              rows[p], gsem[p]).wait()
        pltpu.make_async_copy(dst_hbm.at[pl.ds(wid * EPW + j * B_CH, B_CH)],
                              didx[p], gsem[p]).wait()
        pltpu.make_async_copy(ew_hbm.at[pl.ds(wid * EPW + j * B_CH, B_CH)],
                              ewc[p], gsem[p]).wait()

    def scale(p):
        def group_body(g, _):
            ew16 = ewc[p][pl.ds(g * L, L)]
            base = g * L
            for i in range(L):
                s = ew16[i]
                for k in range(D // L):
                    sl = pl.ds(k * L, L)
                    rows[p][base + i, sl] = rows[p][base + i, sl] * s
            return 0

        lax.fori_loop(0, B_CH // L, group_body, 0)

    def scatter_start(p):
        pltpu.async_copy(rows[p], acc_sh.at[didx[p]], ssem[p], add=True)

    def scatter_wait(p):
        pltpu.make_async_copy(rows[p], acc_sh.at[didx[p]], ssem[p]).wait()

    def step(j, p, wait_prev, next_j):
        gather_wait(j, p)
        scale(p)
        scatter_start(p)
        q = (p + 2) % 3
        if wait_prev:
            scatter_wait(q)
        if next_j is not None:
            gather_start(next_j, q)

    gather_start(0, 0)
    gather_start(1, 1)
    step(0, 0, False, 2)
    step(1, 1, True, 3)

    def ring_body(g, _):
        j = 3 * g + 2
        step(j, 2, True, j + 2)
        step(j + 1, 0, True, j + 3)
        step(j + 2, 1, True, j + 4)
        return 0

    lax.fori_loop(0, (K_CH - 5) // 3, ring_body, 0)
    step(K_CH - 3, 2, True, K_CH - 1)
    step(K_CH - 2, 0, True, None)
    step(K_CH - 1, 1, True, None)
    scatter_wait(1)

    plsc.subcore_barrier()

    for t in range((NZC + NS - 1) // NS):
        c = sid + NS * t

        @pl.when(c < NZC)
        def _():
            pltpu.sync_copy(acc_sh.at[pl.ds(c * ZR, ZR)],
                            out_hbm.at[pl.ds(cid * N_NODES + c * ZR, ZR)])


@functools.cache
def _agg_kernel():
    mesh = plsc.VectorSubcoreMesh(core_axis_name="c", subcore_axis_name="s",
                                  num_cores=NC, num_subcores=NS)
    return pl.kernel(
        _agg_body,
        out_type=jax.ShapeDtypeStruct((NC * N_NODES, D), jnp.float32),
        mesh=mesh,
        compiler_params=pltpu.CompilerParams(needs_layout_passes=False),
        scratch_types=[
            pltpu.VMEM((EPW,), jnp.int32),
            pltpu.VMEM((B_CH,), jnp.int32),
            pltpu.VMEM((B_CH,), jnp.int32),
            pltpu.VMEM((B_CH,), jnp.int32),
            pltpu.VMEM((B_CH,), jnp.float32),
            pltpu.VMEM((B_CH,), jnp.float32),
            pltpu.VMEM((B_CH,), jnp.float32),
            pltpu.VMEM((B_CH, D), jnp.float32),
            pltpu.VMEM((B_CH, D), jnp.float32),
            pltpu.VMEM((B_CH, D), jnp.float32),
            pltpu.VMEM_SHARED((N_NODES, D), jnp.float32),
            pltpu.SemaphoreType.DMA,
            pltpu.SemaphoreType.DMA,
            pltpu.SemaphoreType.DMA,
            pltpu.SemaphoreType.DMA,
            pltpu.SemaphoreType.DMA,
            pltpu.SemaphoreType.DMA,
        ],
    )


BLK = 2000


def _tcA_body(degp_ref, x_ref, w_ref, hp_ref, dinv_ref):
    deg = jnp.sum(degp_ref[...], axis=1) + 2.0
    dinv = jnp.where(deg > 0, lax.rsqrt(jnp.maximum(deg, 1e-30)), 0.0)
    h = jnp.dot(x_ref[...], w_ref[...], preferred_element_type=jnp.float32)
    hp_ref[...] = h * dinv[:, None]
    dinv_ref[...] = dinv[:, None]


_tcA = pl.pallas_call(
    _tcA_body,
    grid=(N_NODES // BLK,),
    in_specs=[
        pl.BlockSpec((BLK, NW), lambda i: (i, 0)),
        pl.BlockSpec((BLK, D), lambda i: (i, 0)),
        pl.BlockSpec((D, D), lambda i: (0, 0)),
    ],
    out_specs=[
        pl.BlockSpec((BLK, D), lambda i: (i, 0)),
        pl.BlockSpec((BLK, 1), lambda i: (i, 0)),
    ],
    out_shape=[
        jax.ShapeDtypeStruct((N_NODES, D), jnp.float32),
        jax.ShapeDtypeStruct((N_NODES, 1), jnp.float32),
    ],
)


def _tcC_body(p_ref, hp_ref, dinv_ref, b1_ref, wfc_ref, bfc_ref, out_ref):
    acc = p_ref[0] + p_ref[1]
    dinv = dinv_ref[...]
    pre = dinv * acc + (2.0 * dinv) * hp_ref[...] + b1_ref[...]
    r = jnp.maximum(pre, 0.0)
    out_ref[...] = jnp.dot(r, wfc_ref[...],
                           preferred_element_type=jnp.float32) + bfc_ref[...]


_tcC = pl.pallas_call(
    _tcC_body,
    grid=(N_NODES // BLK,),
    in_specs=[
        pl.BlockSpec((NC, BLK, D), lambda i: (0, i, 0)),
        pl.BlockSpec((BLK, D), lambda i: (i, 0)),
        pl.BlockSpec((BLK, 1), lambda i: (i, 0)),
        pl.BlockSpec((1, D), lambda i: (0, 0)),
        pl.BlockSpec((D, 1), lambda i: (0, 0)),
        pl.BlockSpec((1, 1), lambda i: (0, 0)),
    ],
    out_specs=pl.BlockSpec((BLK, 1), lambda i: (i, 0)),
    out_shape=jax.ShapeDtypeStruct((N_NODES, 1), jnp.float32),
)


def kernel(x, edge_index, edge_attr, W1, b1, Wfc, bfc):
    ei = edge_index.astype(jnp.int32)
    src = ei[0]
    dst = ei[1]
    ew = edge_attr.astype(jnp.float32)

    degp = _deg_kernel()(dst, ew)
    degp_t = degp.reshape(NW, N_NODES).T

    hp, dinv = _tcA(degp_t, x.astype(jnp.float32), W1)

    parts = _agg_kernel()(src, dst, ew, hp)
    parts = parts.reshape(NC, N_NODES, D)

    out = _tcC(parts, hp, dinv, b1.reshape(1, D), Wfc, bfc.reshape(1, 1))
    return out
